# baseline (speedup 1.0000x reference)
import jax
import jax.numpy as jnp
from jax.experimental import pallas as pl
from jax.experimental.pallas import tpu as pltpu

_D_MODEL = 4096
_N_EXPERTS = 64
_NBIG = 30


def _router_body(x_hbm, w_ref, b_ref, o_hbm,
                 sbuf, mbuf, bbuf, sobuf, mobuf, bobuf,
                 ssem, msem, bsem, sosem, mosem, bosem):
    w16 = w_ref[:].astype(jnp.bfloat16)
    bias = b_ref[:]

    def in_copy(off, rows, buf, slot, sem):
        return pltpu.make_async_copy(
            x_hbm.at[pl.ds(off, rows), :], buf.at[slot], sem.at[slot])

    def out_copy(off, rows, buf, slot, sem):
        return pltpu.make_async_copy(
            buf.at[slot], o_hbm.at[pl.ds(off, rows), :], sem.at[slot])

    def gates(xblk):
        logits = jax.lax.dot_general(
            xblk.astype(jnp.bfloat16), w16,
            (((1,), (1,)), ((), ())),
            preferred_element_type=jnp.float32,
        ) + bias
        m = jnp.max(logits, axis=-1, keepdims=True)
        e = jnp.exp(logits - m)
        return e / jnp.sum(e, axis=-1, keepdims=True)

    in_copy(0, 256, sbuf, 0, ssem).start()
    in_copy(256, 256, sbuf, 1, ssem).start()
    in_copy(512, 512, mbuf, 0, msem).start()
    in_copy(1024, 1024, bbuf, 0, bsem).start()

    in_copy(0, 256, sbuf, 0, ssem).wait()
    sobuf[0] = gates(sbuf[0])
    out_copy(0, 256, sobuf, 0, sosem).start()

    in_copy(256, 256, sbuf, 1, ssem).wait()
    sobuf[1] = gates(sbuf[1])
    out_copy(256, 256, sobuf, 1, sosem).start()

    in_copy(512, 512, mbuf, 0, msem).wait()
    mobuf[0] = gates(mbuf[0])
    out_copy(512, 512, mobuf, 0, mosem).start()

    def step(i, carry):
        off = (i + 1) * 1024
        slot = jax.lax.rem(i, 2)

        @pl.when(i < _NBIG - 1)
        def _prefetch():
            in_copy(off + 1024, 1024, bbuf, jax.lax.rem(i + 1, 2), bsem).start()

        @pl.when(i == _NBIG - 3)
        def _tail_m():
            in_copy(_NBIG * 1024 + 1024, 512, mbuf, 0, msem).start()

        @pl.when(i == _NBIG - 2)
        def _tail_s2():
            in_copy(_NBIG * 1024 + 1536, 256, sbuf, 0, ssem).start()

        @pl.when(i == _NBIG - 1)
        def _tail_s3():
            in_copy(_NBIG * 1024 + 1792, 256, sbuf, 1, ssem).start()

        in_copy(off, 1024, bbuf, slot, bsem).wait()
        p = gates(bbuf[slot])

        @pl.when(i >= 2)
        def _drain_prev():
            out_copy(off - 2048, 1024, bobuf, slot, bosem).wait()

        bobuf[slot] = p
        out_copy(off, 1024, bobuf, slot, bosem).start()
        return carry

    jax.lax.fori_loop(0, _NBIG, step, 0)

    base = _NBIG * 1024 + 1024
    in_copy(base, 512, mbuf, 0, msem).wait()
    mobuf[1] = gates(mbuf[0])
    out_copy(base, 512, mobuf, 1, mosem).start()

    in_copy(base + 512, 256, sbuf, 0, ssem).wait()
    out_copy(0, 256, sobuf, 0, sosem).wait()
    sobuf[0] = gates(sbuf[0])
    out_copy(base + 512, 256, sobuf, 0, sosem).start()

    in_copy(base + 768, 256, sbuf, 1, ssem).wait()
    out_copy(256, 256, sobuf, 1, sosem).wait()
    sobuf[1] = gates(sbuf[1])
    out_copy(base + 768, 256, sobuf, 1, sosem).start()

    out_copy((_NBIG - 1) * 1024, 1024, bobuf, jax.lax.rem(_NBIG - 2, 2), bosem).wait()
    out_copy(_NBIG * 1024, 1024, bobuf, jax.lax.rem(_NBIG - 1, 2), bosem).wait()
    out_copy(512, 512, mobuf, 0, mosem).wait()
    out_copy(base, 512, mobuf, 1, mosem).wait()
    out_copy(base + 512, 256, sobuf, 0, sosem).wait()
    out_copy(base + 768, 256, sobuf, 1, sosem).wait()


def kernel(x, W, b):
    n_tokens = x.shape[0]
    b2 = b.reshape(1, _N_EXPERTS)
    return pl.pallas_call(
        _router_body,
        in_specs=[
            pl.BlockSpec(memory_space=pltpu.MemorySpace.HBM),
            pl.BlockSpec(memory_space=pltpu.MemorySpace.VMEM),
            pl.BlockSpec(memory_space=pltpu.MemorySpace.VMEM),
        ],
        out_specs=pl.BlockSpec(memory_space=pltpu.MemorySpace.HBM),
        out_shape=jax.ShapeDtypeStruct((n_tokens, _N_EXPERTS), jnp.float32),
        scratch_shapes=[
            pltpu.VMEM((2, 256, _D_MODEL), jnp.float32),
            pltpu.VMEM((1, 512, _D_MODEL), jnp.float32),
            pltpu.VMEM((2, 1024, _D_MODEL), jnp.float32),
            pltpu.VMEM((2, 256, _N_EXPERTS), jnp.float32),
            pltpu.VMEM((2, 512, _N_EXPERTS), jnp.float32),
            pltpu.VMEM((2, 1024, _N_EXPERTS), jnp.float32),
            pltpu.SemaphoreType.DMA((2,)),
            pltpu.SemaphoreType.DMA((1,)),
            pltpu.SemaphoreType.DMA((2,)),
            pltpu.SemaphoreType.DMA((2,)),
            pltpu.SemaphoreType.DMA((2,)),
            pltpu.SemaphoreType.DMA((2,)),
        ],
    )(x, W, b2)


# speedup vs baseline: 1.0275x; 1.0275x over previous
import jax
import jax.numpy as jnp
from jax.experimental import pallas as pl
from jax.experimental.pallas import tpu as pltpu

_D_MODEL = 4096
_N_EXPERTS = 64
_HALF = 512
_TILE = 2 * _HALF


def _gates(xblk, w, bias):
    logits = jax.lax.dot_general(
        xblk.astype(jnp.bfloat16), w.astype(jnp.bfloat16),
        (((1,), (1,)), ((), ())),
        preferred_element_type=jnp.float32,
    ) + bias
    m = jnp.max(logits, axis=-1, keepdims=True)
    e = jnp.exp(logits - m)
    return e / jnp.sum(e, axis=-1, keepdims=True)


def _router_body(x1_ref, x2_ref, w_ref, b_ref, o_ref):
    w = w_ref[:]
    bias = b_ref[:]
    o_ref[:_HALF, :] = _gates(x1_ref[:], w, bias)
    o_ref[_HALF:, :] = _gates(x2_ref[:], w, bias)


def kernel(x, W, b):
    n_tokens = x.shape[0]
    b2 = b.reshape(1, _N_EXPERTS)
    return pl.pallas_call(
        _router_body,
        grid=(n_tokens // _TILE,),
        in_specs=[
            pl.BlockSpec((_HALF, _D_MODEL), lambda i: (2 * i, 0)),
            pl.BlockSpec((_HALF, _D_MODEL), lambda i: (2 * i + 1, 0)),
            pl.BlockSpec((_N_EXPERTS, _D_MODEL), lambda i: (0, 0)),
            pl.BlockSpec((1, _N_EXPERTS), lambda i: (0, 0)),
        ],
        out_specs=pl.BlockSpec((_TILE, _N_EXPERTS), lambda i: (i, 0)),
        out_shape=jax.ShapeDtypeStruct((n_tokens, _N_EXPERTS), jnp.float32),
        compiler_params=pltpu.CompilerParams(
            dimension_semantics=("parallel",),
        ),
    )(x, x, W, b2)


# speedup vs baseline: 1.0308x; 1.0032x over previous
import jax
import jax.numpy as jnp
from jax.experimental import pallas as pl
from jax.experimental.pallas import tpu as pltpu

_D_MODEL = 4096
_N_EXPERTS = 64
_TILE = 1024


def _router_body(x_ref, w_ref, b_ref, o_ref):
    logits = jax.lax.dot_general(
        x_ref[:].astype(jnp.bfloat16), w_ref[:].astype(jnp.bfloat16),
        (((1,), (1,)), ((), ())),
        preferred_element_type=jnp.float32,
    ) + b_ref[:]
    m = jnp.max(logits, axis=-1, keepdims=True)
    e = jnp.exp(logits - m)
    o_ref[:] = e / jnp.sum(e, axis=-1, keepdims=True)


def kernel(x, W, b):
    n_tokens = x.shape[0]
    b2 = b.reshape(1, _N_EXPERTS)
    return pl.pallas_call(
        _router_body,
        grid=(n_tokens // _TILE,),
        in_specs=[
            pl.BlockSpec((_TILE, _D_MODEL), lambda i: (i, 0)),
            pl.BlockSpec((_N_EXPERTS, _D_MODEL), lambda i: (0, 0)),
            pl.BlockSpec((1, _N_EXPERTS), lambda i: (0, 0)),
        ],
        out_specs=pl.BlockSpec((_TILE, _N_EXPERTS), lambda i: (i, 0)),
        out_shape=jax.ShapeDtypeStruct((n_tokens, _N_EXPERTS), jnp.float32),
        compiler_params=pltpu.CompilerParams(
            dimension_semantics=("parallel",),
        ),
    )(x, W, b2)
